# Initial kernel scaffold; baseline (speedup 1.0000x reference)
import jax
import jax.numpy as jnp
from jax.experimental import pallas as pl


def kernel(output, target, mask):
    raise NotImplementedError("write your pallas kernel here")


# speedup vs baseline: 1.0179x; 1.0179x over previous
import jax
import jax.numpy as jnp
from jax.experimental import pallas as pl
from jax.experimental.pallas import tpu as pltpu

ROWS = 100000
COLS = 512
BLOCK_ROWS = 2000
NUM_BLOCKS = ROWS // BLOCK_ROWS
N_TOTAL = float(ROWS * COLS)


def _body(o_ref, t_ref, m_ref, loss_ref, acc_ref):
    i = pl.program_id(0)

    d = o_ref[...] - t_ref[...]
    d2 = d * d
    m = m_ref[...].astype(jnp.float32)

    psum_m = jnp.sum(d2 * m)
    psum_tot = jnp.sum(d2)
    pcnt = jnp.sum(m)

    @pl.when(i == 0)
    def _init():
        acc_ref[0] = 0.0
        acc_ref[1] = 0.0
        acc_ref[2] = 0.0

    acc_ref[0] += psum_m
    acc_ref[1] += psum_tot
    acc_ref[2] += pcnt

    @pl.when(i == NUM_BLOCKS - 1)
    def _final():
        s_m = acc_ref[0]
        s_tot = acc_ref[1]
        c = acc_ref[2]
        loss = s_m / jnp.maximum(c, 1.0) + (s_tot - s_m) / jnp.maximum(
            N_TOTAL - c, 1.0
        )
        loss_ref[0, 0] = loss


def kernel(output, target, mask):
    loss = pl.pallas_call(
        _body,
        grid=(NUM_BLOCKS,),
        in_specs=[
            pl.BlockSpec((BLOCK_ROWS, COLS), lambda i: (i, 0)),
            pl.BlockSpec((BLOCK_ROWS, COLS), lambda i: (i, 0)),
            pl.BlockSpec((BLOCK_ROWS, COLS), lambda i: (i, 0)),
        ],
        out_specs=pl.BlockSpec(
            (1, 1), lambda i: (0, 0), memory_space=pltpu.SMEM
        ),
        out_shape=jax.ShapeDtypeStruct((1, 1), jnp.float32),
        scratch_shapes=[pltpu.SMEM((3,), jnp.float32)],
    )(output, target, mask)
    return loss[0, 0]
